# baseline (speedup 1.0000x reference)
import jax
import jax.numpy as jnp
from jax import lax
from jax.experimental import pallas as pl
from jax.experimental.pallas import tpu as pltpu
from jax.experimental.pallas import tpu_sc as plsc

_DIM = 64
_N = 1000000
_K = 16384
_NC = 2
_NS = 16
_NW = _NC * _NS
_CHUNK = _K // _NW
_L = 16


def _body(bank_hbm, idx_hbm, out_hbm, idx_v, blk_v, out_v, sem):
    wid = lax.axis_index("s") * _NC + lax.axis_index("c")
    base = wid * _CHUNK
    pltpu.sync_copy(idx_hbm.at[pl.ds(base, _CHUNK)], idx_v)

    iota = lax.iota(jnp.int32, _L)

    @pl.loop(0, _CHUNK, step=_L)
    def _(i):
        v = idx_v[pl.ds(i, _L)]
        for j in range(_L):
            c = lax.shift_right_logical(v[j], 7)
            lane = lax.bitwise_and(v[j], 127)
            pltpu.sync_copy(bank_hbm.at[:, pl.ds(c * 128, 128)], blk_v)
            lanes = jnp.full((_L,), lane, jnp.int32)
            for t in range(_DIM // _L):
                rows = iota + (t * _L)
                col = plsc.load_gather(blk_v, [rows, lanes])
                plsc.store_scatter(out_v, [rows * _CHUNK + (i + j)], col)

    @pl.loop(0, _DIM)
    def _(d):
        pltpu.sync_copy(
            out_v.at[pl.ds(d * _CHUNK, _CHUNK)],
            out_hbm.at[d].at[pl.ds(base, _CHUNK)],
        )


def kernel(membank, n_index):
    mesh = plsc.VectorSubcoreMesh(core_axis_name="c", subcore_axis_name="s")
    gathered = pl.kernel(
        _body,
        out_type=jax.ShapeDtypeStruct((_DIM, _K), jnp.float32),
        mesh=mesh,
        compiler_params=pltpu.CompilerParams(needs_layout_passes=False),
        scratch_types=[
            pltpu.VMEM((_CHUNK,), jnp.int32),
            pltpu.VMEM((_DIM, 128), jnp.float32),
            pltpu.VMEM((_DIM * _CHUNK,), jnp.float32),
            pltpu.SemaphoreType.DMA,
        ],
    )
    return gathered(membank, n_index)


# speedup vs baseline: 2.0994x; 2.0994x over previous
import jax
import jax.numpy as jnp
from jax import lax
from jax.experimental import pallas as pl
from jax.experimental.pallas import tpu as pltpu
from jax.experimental.pallas import tpu_sc as plsc

_DIM = 64
_N = 1000000
_K = 16384
_NC = 2
_NS = 16
_NW = _NC * _NS
_CHUNK = _K // _NW
_L = 16


_DEPTH = 4


def _body(bank_hbm, idx_hbm, out_hbm, idx_v,
          blk0, blk1, blk2, blk3, out_v, sem0, sem1, sem2, sem3):
    wid = lax.axis_index("s") * _NC + lax.axis_index("c")
    base = wid * _CHUNK
    pltpu.sync_copy(idx_hbm.at[pl.ds(base, _CHUNK)], idx_v)

    iota = lax.iota(jnp.int32, _L)
    bufs = (blk0, blk1, blk2, blk3)
    sems = (sem0, sem1, sem2, sem3)

    @pl.loop(0, _CHUNK, step=_L)
    def _(i):
        v = idx_v[pl.ds(i, _L)]
        cs = [lax.shift_right_logical(v[j], 7) for j in range(_L)]
        lanes = [lax.bitwise_and(v[j], 127) for j in range(_L)]

        def start(j):
            pltpu.make_async_copy(
                bank_hbm.at[:, pl.ds(cs[j] * 128, 128)],
                bufs[j % _DEPTH], sems[j % _DEPTH],
            ).start()

        def finish(j):
            blk = bufs[j % _DEPTH]
            pltpu.make_async_copy(
                bank_hbm.at[:, pl.ds(cs[j] * 128, 128)],
                blk, sems[j % _DEPTH],
            ).wait()
            lv = jnp.full((_L,), lanes[j], jnp.int32)
            for t in range(_DIM // _L):
                rows = iota + (t * _L)
                col = plsc.load_gather(blk, [rows, lv])
                plsc.store_scatter(out_v, [rows * _CHUNK + (i + j)], col)

        for j in range(_L):
            if j >= _DEPTH:
                finish(j - _DEPTH)
            start(j)
        for j in range(_L - _DEPTH, _L):
            finish(j)

    @pl.loop(0, _DIM)
    def _(d):
        pltpu.sync_copy(
            out_v.at[pl.ds(d * _CHUNK, _CHUNK)],
            out_hbm.at[d].at[pl.ds(base, _CHUNK)],
        )


def kernel(membank, n_index):
    mesh = plsc.VectorSubcoreMesh(core_axis_name="c", subcore_axis_name="s")
    gathered = pl.kernel(
        _body,
        out_type=jax.ShapeDtypeStruct((_DIM, _K), jnp.float32),
        mesh=mesh,
        compiler_params=pltpu.CompilerParams(needs_layout_passes=False),
        scratch_types=[
            pltpu.VMEM((_CHUNK,), jnp.int32),
            pltpu.VMEM((_DIM, 128), jnp.float32),
            pltpu.VMEM((_DIM, 128), jnp.float32),
            pltpu.VMEM((_DIM, 128), jnp.float32),
            pltpu.VMEM((_DIM, 128), jnp.float32),
            pltpu.VMEM((_DIM * _CHUNK,), jnp.float32),
            pltpu.SemaphoreType.DMA,
            pltpu.SemaphoreType.DMA,
            pltpu.SemaphoreType.DMA,
            pltpu.SemaphoreType.DMA,
        ],
    )
    return gathered(membank, n_index)


# speedup vs baseline: 2.3427x; 1.1159x over previous
import jax
import jax.numpy as jnp
from jax import lax
from jax.experimental import pallas as pl
from jax.experimental.pallas import tpu as pltpu
from jax.experimental.pallas import tpu_sc as plsc

_DIM = 64
_N = 1000000
_K = 16384
_NC = 2
_NS = 16
_NW = _NC * _NS
_CHUNK = _K // _NW
_L = 16


_DEPTH = 8


def _body(bank_hbm, idx_hbm, out_hbm, idx_v, *scratch):
    bufs = scratch[:_DEPTH]
    out_v = scratch[_DEPTH]
    sems = scratch[_DEPTH + 1:]
    wid = lax.axis_index("s") * _NC + lax.axis_index("c")
    base = wid * _CHUNK
    pltpu.sync_copy(idx_hbm.at[pl.ds(base, _CHUNK)], idx_v)

    iota = lax.iota(jnp.int32, _L)

    @pl.loop(0, _CHUNK, step=_L)
    def _(i):
        v = idx_v[pl.ds(i, _L)]
        cs = [lax.shift_right_logical(v[j], 7) for j in range(_L)]
        lanes = [lax.bitwise_and(v[j], 127) for j in range(_L)]

        def start(j):
            pltpu.make_async_copy(
                bank_hbm.at[:, pl.ds(cs[j] * 128, 128)],
                bufs[j % _DEPTH], sems[j % _DEPTH],
            ).start()

        def finish(j):
            blk = bufs[j % _DEPTH]
            pltpu.make_async_copy(
                bank_hbm.at[:, pl.ds(cs[j] * 128, 128)],
                blk, sems[j % _DEPTH],
            ).wait()
            lv = jnp.full((_L,), lanes[j], jnp.int32)
            for t in range(_DIM // _L):
                rows = iota + (t * _L)
                col = plsc.load_gather(blk, [rows, lv])
                plsc.store_scatter(out_v, [rows * _CHUNK + (i + j)], col)

        for j in range(_L):
            if j >= _DEPTH:
                finish(j - _DEPTH)
            start(j)
        for j in range(_L - _DEPTH, _L):
            finish(j)

    @pl.loop(0, _DIM)
    def _(d):
        pltpu.sync_copy(
            out_v.at[pl.ds(d * _CHUNK, _CHUNK)],
            out_hbm.at[d].at[pl.ds(base, _CHUNK)],
        )


def kernel(membank, n_index):
    mesh = plsc.VectorSubcoreMesh(core_axis_name="c", subcore_axis_name="s")
    gathered = pl.kernel(
        _body,
        out_type=jax.ShapeDtypeStruct((_DIM, _K), jnp.float32),
        mesh=mesh,
        compiler_params=pltpu.CompilerParams(needs_layout_passes=False),
        scratch_types=(
            [pltpu.VMEM((_CHUNK,), jnp.int32)]
            + [pltpu.VMEM((_DIM, 128), jnp.float32)] * _DEPTH
            + [pltpu.VMEM((_DIM * _CHUNK,), jnp.float32)]
            + [pltpu.SemaphoreType.DMA] * _DEPTH
        ),
    )
    return gathered(membank, n_index)


# speedup vs baseline: 2.6639x; 1.1371x over previous
import jax
import jax.numpy as jnp
from jax import lax
from jax.experimental import pallas as pl
from jax.experimental.pallas import tpu as pltpu
from jax.experimental.pallas import tpu_sc as plsc

_DIM = 64
_N = 1000000
_K = 16384
_NC = 2
_NS = 16
_NW = _NC * _NS
_CHUNK = _K // _NW
_L = 16


_DEPTH = 8


def _body(bank_hbm, idx_hbm, out_hbm, idx_v, *scratch):
    bufs = scratch[:_DEPTH]
    out_v = scratch[_DEPTH]
    sems = scratch[_DEPTH + 1:]
    wid = lax.axis_index("s") * _NC + lax.axis_index("c")
    base = wid * _CHUNK
    pltpu.sync_copy(idx_hbm.at[pl.ds(base, _CHUNK)], idx_v)

    iota = lax.iota(jnp.int32, _L)

    def _start(c, slot):
        pltpu.make_async_copy(
            bank_hbm.at[:, pl.ds(c * 128, 128)], bufs[slot], sems[slot]
        ).start()

    def _finish(c, lane, pos, slot):
        blk = bufs[slot]
        pltpu.make_async_copy(
            bank_hbm.at[:, pl.ds(c * 128, 128)], blk, sems[slot]
        ).wait()
        lv = jnp.full((_L,), lane, jnp.int32)
        for t in range(_DIM // _L):
            rows = iota + (t * _L)
            col = plsc.load_gather(blk, [rows, lv])
            plsc.store_scatter(out_v, [rows * _CHUNK + pos], col)

    @pl.loop(0, _CHUNK, step=_L)
    def _(i):
        v = idx_v[pl.ds(i, _L)]
        cs = [lax.shift_right_logical(v[j], 7) for j in range(_L)]
        lanes = [lax.bitwise_and(v[j], 127) for j in range(_L)]
        ip = jnp.maximum(i - _L, 0)
        vp = idx_v[pl.ds(ip, _L)]
        csp = [lax.shift_right_logical(vp[j], 7) for j in range(_L)]
        lanesp = [lax.bitwise_and(vp[j], 127) for j in range(_L)]

        for j in range(_L):
            m = j - _DEPTH
            if m >= 0:
                _finish(cs[m], lanes[m], i + m, m % _DEPTH)
            else:
                jj = j + _L - _DEPTH

                @pl.when(i > 0)
                def _():
                    _finish(csp[jj], lanesp[jj], ip + jj, jj % _DEPTH)

            _start(cs[j], j % _DEPTH)

    iL = _CHUNK - _L
    vl = idx_v[pl.ds(iL, _L)]
    for jj in range(_L - _DEPTH, _L):
        c = lax.shift_right_logical(vl[jj], 7)
        lane = lax.bitwise_and(vl[jj], 127)
        _finish(c, lane, iL + jj, jj % _DEPTH)

    @pl.loop(0, _DIM)
    def _(d):
        pltpu.sync_copy(
            out_v.at[pl.ds(d * _CHUNK, _CHUNK)],
            out_hbm.at[d].at[pl.ds(base, _CHUNK)],
        )


def kernel(membank, n_index):
    mesh = plsc.VectorSubcoreMesh(core_axis_name="c", subcore_axis_name="s")
    gathered = pl.kernel(
        _body,
        out_type=jax.ShapeDtypeStruct((_DIM, _K), jnp.float32),
        mesh=mesh,
        compiler_params=pltpu.CompilerParams(needs_layout_passes=False),
        scratch_types=(
            [pltpu.VMEM((_CHUNK,), jnp.int32)]
            + [pltpu.VMEM((_DIM, 128), jnp.float32)] * _DEPTH
            + [pltpu.VMEM((_DIM * _CHUNK,), jnp.float32)]
            + [pltpu.SemaphoreType.DMA] * _DEPTH
        ),
    )
    return gathered(membank, n_index)
